# baseline (speedup 1.0000x reference)
import functools

import jax
import jax.numpy as jnp
from jax import lax
from jax.experimental import pallas as pl
from jax.experimental.pallas import tpu as pltpu
from jax.experimental.pallas import tpu_sc as plsc

_NC = 2
_NS = 16
_NW = _NC * _NS
_L = 16
_BP = 17


@functools.lru_cache(maxsize=None)
def _make_gather(V, D, B, H):
    b_per_w = B // _NW
    bchunk = _L
    n_chunks = b_per_w // bchunk
    chunk = bchunk * H
    mesh = plsc.VectorSubcoreMesh(core_axis_name="c", subcore_axis_name="s")

    @functools.partial(
        pl.kernel,
        mesh=mesh,
        out_type=jax.ShapeDtypeStruct((H, D, B), jnp.float32),
        scratch_types=[
            pltpu.VMEM((chunk,), jnp.int32),
            pltpu.VMEM((chunk,), jnp.int32),
            pltpu.VMEM((chunk, D), jnp.float32),
            pltpu.VMEM((chunk, D), jnp.float32),
            pltpu.VMEM((H, D, _BP), jnp.float32),
            pltpu.VMEM((H, D, _BP), jnp.float32),
            pltpu.SemaphoreType.DMA,
            pltpu.SemaphoreType.DMA,
            pltpu.SemaphoreType.DMA,
            pltpu.SemaphoreType.DMA,
            pltpu.SemaphoreType.DMA,
            pltpu.SemaphoreType.DMA,
        ],
        compiler_params=pltpu.CompilerParams(
            use_tc_tiling_on_sc=False, needs_layout_passes=False),
    )
    def k(table_hbm, idx_hbm, out_hbm, i0, i1, g0, g1, t0, t1,
          isem0, isem1, gsem0, gsem1, wsem0, wsem1):
        wid = lax.axis_index("s") * _NC + lax.axis_index("c")
        b_base = wid * b_per_w
        r_base = b_base * H

        ibufs = (i0, i1)
        gbufs = (g0, g1)
        tbufs = (t0, t1)
        isems = (isem0, isem1)
        gsems = (gsem0, gsem1)
        wsems = (wsem0, wsem1)

        def fire_idx(c, p):
            pltpu.async_copy(
                idx_hbm.at[pl.ds(r_base + c * chunk, chunk)],
                ibufs[p], isems[p])

        def wait_idx(p):
            pltpu.make_async_copy(
                idx_hbm.at[pl.ds(0, chunk)], ibufs[p], isems[p]).wait()

        def fire_gather(p):
            pltpu.async_copy(table_hbm.at[ibufs[p]], gbufs[p], gsems[p])

        def wait_gather(p):
            pltpu.make_async_copy(
                table_hbm.at[pl.ds(0, chunk)], gbufs[p], gsems[p]).wait()

        def drain_write(p):
            pltpu.make_async_copy(
                table_hbm.at[pl.ds(0, chunk)], gbufs[p], wsems[p]).wait()

        def transpose(p):
            gb = gbufs[p]
            tb = tbufs[p]
            dvecs = [
                (jnp.full((_L,), d0, jnp.int32) + lax.iota(jnp.int32, _L))
                for d0 in range(0, D, _L)
            ]
            zeros = jnp.zeros((_L,), jnp.int32)

            @plsc.parallel_loop(0, H, unroll=3)
            def hbody(h):
                hvec = zeros + h
                for b in range(bchunk):
                    for f, d0 in enumerate(range(0, D, _L)):
                        vals = gb[b * H + h, pl.ds(d0, _L)]
                        plsc.store_scatter(
                            tb, [hvec, dvecs[f], zeros + b], vals)

        def fire_write(c, p):
            pltpu.async_copy(
                tbufs[p].at[:, :, pl.ds(0, bchunk)],
                out_hbm.at[:, :, pl.ds(b_base + c * bchunk, bchunk)],
                wsems[p])

        fire_idx(0, 0)
        fire_idx(1, 1)
        wait_idx(0)
        fire_gather(0)
        wait_idx(1)
        fire_gather(1)
        n_pairs = n_chunks // 2

        def pair(j, _):
            c0 = 2 * j
            c1 = c0 + 1

            wait_gather(0)

            @pl.when(j + 1 < n_pairs)
            def _():
                fire_idx(c0 + 2, 0)

            @pl.when(j >= 1)
            def _():
                drain_write(0)
            transpose(0)

            @pl.when(j + 1 < n_pairs)
            def _():
                wait_idx(0)
                fire_gather(0)
            fire_write(c0, 0)

            wait_gather(1)

            @pl.when(j + 1 < n_pairs)
            def _():
                fire_idx(c1 + 2, 1)

            @pl.when(j >= 1)
            def _():
                drain_write(1)
            transpose(1)

            @pl.when(j + 1 < n_pairs)
            def _():
                wait_idx(1)
                fire_gather(1)
            fire_write(c1, 1)
            return 0

        lax.fori_loop(0, n_pairs, pair, 0)
        drain_write(0)
        drain_write(1)

    return k


def kernel(table, inputs):
    B, H = inputs.shape
    V, D = table.shape
    idx = inputs.reshape(B * H).astype(jnp.int32)
    out_t = _make_gather(V, D, B, H)(table, idx)
    return out_t.transpose(2, 0, 1)


# speedup vs baseline: 1.0703x; 1.0703x over previous
import functools

import jax
import jax.numpy as jnp
from jax import lax
from jax.experimental import pallas as pl
from jax.experimental.pallas import tpu as pltpu
from jax.experimental.pallas import tpu_sc as plsc

_NC = 2
_NS = 16
_NW = _NC * _NS
_L = 16
_BP = 17


@functools.lru_cache(maxsize=None)
def _make_gather(V, D, B, H):
    b_per_w = B // _NW
    bchunk = _L
    n_chunks = b_per_w // bchunk
    chunk = bchunk * H
    mesh = plsc.VectorSubcoreMesh(core_axis_name="c", subcore_axis_name="s")

    @functools.partial(
        pl.kernel,
        mesh=mesh,
        out_type=jax.ShapeDtypeStruct((H, D, B), jnp.float32),
        scratch_types=[
            pltpu.VMEM((chunk,), jnp.int32),
            pltpu.VMEM((chunk,), jnp.int32),
            pltpu.VMEM((chunk, D), jnp.float32),
            pltpu.VMEM((chunk, D), jnp.float32),
            pltpu.VMEM((H, D, _BP), jnp.float32),
            pltpu.VMEM((H, D, _BP), jnp.float32),
            pltpu.SemaphoreType.DMA,
            pltpu.SemaphoreType.DMA,
            pltpu.SemaphoreType.DMA,
            pltpu.SemaphoreType.DMA,
            pltpu.SemaphoreType.DMA,
            pltpu.SemaphoreType.DMA,
        ],
        compiler_params=pltpu.CompilerParams(
            use_tc_tiling_on_sc=False, needs_layout_passes=False),
    )
    def k(table_hbm, idx_hbm, out_hbm, i0, i1, g0, g1, t0, t1,
          isem0, isem1, gsem0, gsem1, wsem0, wsem1):
        wid = lax.axis_index("s") * _NC + lax.axis_index("c")
        b_base = wid * b_per_w
        r_base = b_base * H

        ibufs = (i0, i1)
        gbufs = (g0, g1)
        tbufs = (t0, t1)
        isems = (isem0, isem1)
        gsems = (gsem0, gsem1)
        wsems = (wsem0, wsem1)

        def fire_idx(c, p):
            pltpu.async_copy(
                idx_hbm.at[pl.ds(r_base + c * chunk, chunk)],
                ibufs[p], isems[p])

        def wait_idx(p):
            pltpu.make_async_copy(
                idx_hbm.at[pl.ds(0, chunk)], ibufs[p], isems[p]).wait()

        def fire_gather(p):
            pltpu.async_copy(table_hbm.at[ibufs[p]], gbufs[p], gsems[p])

        def wait_gather(p):
            pltpu.make_async_copy(
                table_hbm.at[pl.ds(0, chunk)], gbufs[p], gsems[p]).wait()

        def drain_write(p):
            pltpu.make_async_copy(
                table_hbm.at[pl.ds(0, chunk)], gbufs[p], wsems[p]).wait()

        def transpose(p):
            gb = gbufs[p]
            tb = tbufs[p]
            dvecs = [
                (jnp.full((_L,), d0, jnp.int32) + lax.iota(jnp.int32, _L))
                for d0 in range(0, D, _L)
            ]
            zeros = jnp.zeros((_L,), jnp.int32)

            @plsc.parallel_loop(0, H, unroll=2)
            def hbody(h):
                hvec = zeros + h
                for b in range(bchunk):
                    for f, d0 in enumerate(range(0, D, _L)):
                        vals = gb[b * H + h, pl.ds(d0, _L)]
                        plsc.store_scatter(
                            tb, [hvec, dvecs[f], zeros + b], vals)

        def fire_write(c, p):
            pltpu.async_copy(
                tbufs[p].at[:, :, pl.ds(0, bchunk)],
                out_hbm.at[:, :, pl.ds(b_base + c * bchunk, bchunk)],
                wsems[p])

        fire_idx(0, 0)
        fire_idx(1, 1)
        wait_idx(0)
        fire_gather(0)
        wait_idx(1)
        fire_gather(1)
        n_pairs = n_chunks // 2

        def pair(j, _):
            c0 = 2 * j
            c1 = c0 + 1

            wait_gather(0)

            @pl.when(j + 1 < n_pairs)
            def _():
                fire_idx(c0 + 2, 0)

            @pl.when(j >= 1)
            def _():
                drain_write(0)
            transpose(0)

            @pl.when(j + 1 < n_pairs)
            def _():
                wait_idx(0)
                fire_gather(0)
            fire_write(c0, 0)

            wait_gather(1)

            @pl.when(j + 1 < n_pairs)
            def _():
                fire_idx(c1 + 2, 1)

            @pl.when(j >= 1)
            def _():
                drain_write(1)
            transpose(1)

            @pl.when(j + 1 < n_pairs)
            def _():
                wait_idx(1)
                fire_gather(1)
            fire_write(c1, 1)
            return 0

        lax.fori_loop(0, n_pairs, pair, 0)
        drain_write(0)
        drain_write(1)

    return k


def kernel(table, inputs):
    B, H = inputs.shape
    V, D = table.shape
    idx = inputs.reshape(B * H).astype(jnp.int32)
    out_t = _make_gather(V, D, B, H)(table, idx)
    return out_t.transpose(2, 0, 1)
